# Initial kernel scaffold; baseline (speedup 1.0000x reference)
import jax
import jax.numpy as jnp
from jax.experimental import pallas as pl


def kernel(x, table):
    raise NotImplementedError("write your pallas kernel here")


# speedup vs baseline: 1.0220x; 1.0220x over previous
import jax
import jax.numpy as jnp
from jax import lax
from jax.experimental import pallas as pl
from jax.experimental.pallas import tpu as pltpu
from jax.experimental.pallas import tpu_sc as plsc

NC, NS = 2, 16
NW = NC * NS
BATCH = 16384
HIST = 50
EMBED = 32
TOTAL = BATCH * HIST
PER_W = TOTAL // NW
CHUNK = 128
NCH = PER_W // CHUNK


def _body(table_hbm, idx_hbm, out_hbm, idx_v, rows_v, gsem):
    wid = lax.axis_index("s") * NC + lax.axis_index("c")
    base = wid * PER_W
    pltpu.sync_copy(idx_hbm.at[pl.ds(base, PER_W)], idx_v)

    def step(j, _):
        off = j * CHUNK
        pltpu.async_copy(
            table_hbm.at[idx_v.at[pl.ds(off, CHUNK)]], rows_v, gsem
        ).wait()
        pltpu.sync_copy(rows_v, out_hbm.at[pl.ds(base + off, CHUNK)])
        return 0

    lax.fori_loop(0, NCH, step, 0)


@jax.jit
def _gather(xflat, table):
    mesh = plsc.VectorSubcoreMesh(
        core_axis_name="c", subcore_axis_name="s",
        num_cores=NC, num_subcores=NS,
    )
    return pl.kernel(
        _body,
        out_type=jax.ShapeDtypeStruct((TOTAL, EMBED), jnp.float32),
        mesh=mesh,
        scratch_types=[
            pltpu.VMEM((PER_W,), jnp.int32),
            pltpu.VMEM((CHUNK, EMBED), jnp.float32),
            pltpu.SemaphoreType.DMA,
        ],
        compiler_params=pltpu.CompilerParams(use_tc_tiling_on_sc=False),
    )(table, xflat)


def kernel(x, table):
    xflat = x.reshape(-1)
    out = _gather(xflat, table)
    return out.reshape(x.shape + (EMBED,))


# speedup vs baseline: 1.1118x; 1.0879x over previous
import jax
import jax.numpy as jnp
from jax import lax
from jax.experimental import pallas as pl
from jax.experimental.pallas import tpu as pltpu
from jax.experimental.pallas import tpu_sc as plsc

NC, NS = 2, 16
NW = NC * NS
BATCH = 16384
HIST = 50
EMBED = 32
TOTAL = BATCH * HIST
PER_W = TOTAL // NW
CHUNK = 128
K = 5
GK = K * CHUNK
NBUF = 4
G = PER_W // GK


def _body(table_hbm, idx_hbm, out_hbm, idx_v, rows_v, gsem, osem):
    wid = lax.axis_index("s") * NC + lax.axis_index("c")
    base = wid * PER_W
    pltpu.sync_copy(idx_hbm.at[pl.ds(base, PER_W)], idx_v)

    def fire(g, b):
        goff = g * GK
        for k in range(K):
            pltpu.async_copy(
                table_hbm.at[idx_v.at[pl.ds(goff + k * CHUNK, CHUNK)]],
                rows_v.at[b, pl.ds(k * CHUNK, CHUNK)],
                gsem.at[b],
            )

    def turn(g, b):
        goff = g * GK
        for k in range(K):
            pltpu.make_async_copy(
                table_hbm.at[idx_v.at[pl.ds(goff + k * CHUNK, CHUNK)]],
                rows_v.at[b, pl.ds(k * CHUNK, CHUNK)],
                gsem.at[b],
            ).wait()
        cp = pltpu.async_copy(
            rows_v.at[b], out_hbm.at[pl.ds(base + g * GK, GK)], osem.at[b]
        )
        cp.wait()

        @pl.when(g + NBUF < G)
        def _():
            fire(g + NBUF, b)

    for b in range(NBUF):
        fire(b, b)

    def step(i, _):
        g = i * NBUF
        for b in range(NBUF):
            turn(g + b, b)
        return 0

    lax.fori_loop(0, G // NBUF, step, 0)


@jax.jit
def _gather(xflat, table):
    mesh = plsc.VectorSubcoreMesh(
        core_axis_name="c", subcore_axis_name="s",
        num_cores=NC, num_subcores=NS,
    )
    return pl.kernel(
        _body,
        out_type=jax.ShapeDtypeStruct((TOTAL, EMBED), jnp.float32),
        mesh=mesh,
        scratch_types=[
            pltpu.VMEM((PER_W,), jnp.int32),
            pltpu.VMEM((NBUF, GK, EMBED), jnp.float32),
            pltpu.SemaphoreType.DMA((NBUF,)),
            pltpu.SemaphoreType.DMA((NBUF,)),
        ],
        compiler_params=pltpu.CompilerParams(use_tc_tiling_on_sc=False),
    )(table, xflat)


def kernel(x, table):
    xflat = x.reshape(-1)
    out = _gather(xflat, table)
    return out.reshape(x.shape + (EMBED,))


# speedup vs baseline: 1.5687x; 1.4110x over previous
import jax
import jax.numpy as jnp
from jax import lax
from jax.experimental import pallas as pl
from jax.experimental.pallas import tpu as pltpu
from jax.experimental.pallas import tpu_sc as plsc

NC, NS = 2, 16
NW = NC * NS
BATCH = 16384
HIST = 50
EMBED = 32
VOCAB = 1000000
SRW = 128
NSR = VOCAB * EMBED // SRW
NB = 128
NBB = BATCH // NB
BPW = NBB // NW
RH = 10
NHG = HIST // RH


def _body(tbl_hbm, xT_hbm, out_hbm, idx_v, sr_v, gb_v, slab_v, gsem):
    wid = lax.axis_index("s") * NC + lax.axis_index("c")
    lane = lax.iota(jnp.int32, 16)

    def fire(h, buf):
        pltpu.async_copy(tbl_hbm.at[sr_v.at[h]], gb_v.at[buf], gsem.at[buf])

    def wait(h, buf):
        pltpu.make_async_copy(
            tbl_hbm.at[sr_v.at[h]], gb_v.at[buf], gsem.at[buf]
        ).wait()

    def extract(i, h, buf):
        def chunk(j16, _):
            dr16 = idx_v[h, pl.ds(j16 * 16, 16)]
            jb = j16 * 16 + lane
            for e in range(EMBED):
                vals = plsc.load_gather(gb_v.at[buf], [jb, dr16 + e])
                slab_v[i, e, pl.ds(j16 * 16, 16)] = vals
            return 0

        lax.fori_loop(0, NB // 16, chunk, 0)

    def do_block(k, _):
        b0 = (wid * BPW + k) * NB
        pltpu.sync_copy(xT_hbm.at[:, pl.ds(b0, NB)], idx_v)

        def prep(i, _):
            row = i // 8
            c = (i % 8) * 16
            v = idx_v[row, pl.ds(c, 16)]
            sr_v[row, pl.ds(c, 16)] = lax.shift_right_logical(v, 2)
            idx_v[row, pl.ds(c, 16)] = lax.shift_left(
                lax.bitwise_and(v, 3), 5
            )
            return 0

        lax.fori_loop(0, HIST * (NB // 16), prep, 0)

        fire(0, 0)

        def do_slab(hg, _):
            h0 = hg * RH

            def pair(i2, _):
                h = h0 + i2 * 2
                wait(h, 0)

                @pl.when(h + 1 < HIST)
                def _():
                    fire(h + 1, 1)

                extract(i2 * 2, h, 0)
                wait(h + 1, 1)

                @pl.when(h + 2 < HIST)
                def _():
                    fire(h + 2, 0)

                extract(i2 * 2 + 1, h + 1, 1)
                return 0

            lax.fori_loop(0, RH // 2, pair, 0)
            pltpu.sync_copy(
                slab_v, out_hbm.at[pl.ds(h0, RH), :, pl.ds(b0, NB)]
            )
            return 0

        lax.fori_loop(0, NHG, do_slab, 0)
        return 0

    lax.fori_loop(0, BPW, do_block, 0)


@jax.jit
def _gather(xT, tblS):
    mesh = plsc.VectorSubcoreMesh(
        core_axis_name="c", subcore_axis_name="s",
        num_cores=NC, num_subcores=NS,
    )
    return pl.kernel(
        _body,
        out_type=jax.ShapeDtypeStruct((HIST, EMBED, BATCH), jnp.float32),
        mesh=mesh,
        scratch_types=[
            pltpu.VMEM((HIST, NB), jnp.int32),
            pltpu.VMEM((HIST, NB), jnp.int32),
            pltpu.VMEM((2, NB, SRW), jnp.float32),
            pltpu.VMEM((RH, EMBED, NB), jnp.float32),
            pltpu.SemaphoreType.DMA((2,)),
        ],
        compiler_params=pltpu.CompilerParams(
            use_tc_tiling_on_sc=True, needs_layout_passes=False
        ),
    )(tblS, xT)


def kernel(x, table):
    tblS = table.reshape(NSR, SRW)
    xT = x.T
    outT = _gather(xT, tblS)
    return jnp.transpose(outT, (2, 0, 1))
